# baseline (speedup 1.0000x reference)
import jax
import jax.numpy as jnp
from jax import lax
from jax.experimental import pallas as pl
from jax.experimental.pallas import tpu as pltpu
from jax.experimental.pallas import tpu_sc as plsc

BATCH = 4096
HIST = 200
EMBED_DIM = 64

NC = 2
NS = 16
NW = NC * NS
ROWS_W = BATCH // NW


def _gather_body(words_hbm, table_hbm, out_hbm, idx_v, rows_v, gsems, osems):
    wid = lax.axis_index("s") * NC + lax.axis_index("c")
    base = wid * ROWS_W
    pltpu.sync_copy(words_hbm.at[pl.ds(base, ROWS_W)], idx_v)

    def fire_gather(i, h):
        pltpu.async_copy(
            table_hbm.at[idx_v.at[i]],
            rows_v.at[h],
            gsems[h],
        )

    def drain_gather(h):
        pltpu.make_async_copy(
            table_hbm.at[pl.ds(0, HIST)], rows_v.at[h], gsems[h]
        ).wait()

    def fire_write(i, h):
        pltpu.async_copy(rows_v.at[h], out_hbm.at[base + i], osems[h])

    def drain_write(h):
        pltpu.make_async_copy(rows_v.at[h], out_hbm.at[base], osems[h]).wait()

    fire_gather(0, 0)

    def body(t, _):
        for h in (0, 1):
            i = 2 * t + h
            @pl.when(jnp.logical_and(i >= 1, i + 1 < ROWS_W))
            def _():
                drain_write(1 - h)

            @pl.when(i + 1 < ROWS_W)
            def _():
                fire_gather(i + 1, 1 - h)

            drain_gather(h)
            fire_write(i, h)
        return ()

    lax.fori_loop(0, ROWS_W // 2, body, (), unroll=False)
    drain_write(0)
    drain_write(1)


@jax.jit
def kernel(words, table):
    mesh = plsc.VectorSubcoreMesh(core_axis_name="c", subcore_axis_name="s")
    return pl.kernel(
        _gather_body,
        out_type=jax.ShapeDtypeStruct((BATCH, HIST, EMBED_DIM), jnp.float32),
        mesh=mesh,
        scratch_types=[
            pltpu.VMEM((ROWS_W, HIST), jnp.int32),
            pltpu.VMEM((2, HIST, EMBED_DIM), jnp.float32),
            [pltpu.SemaphoreType.DMA, pltpu.SemaphoreType.DMA],
            [pltpu.SemaphoreType.DMA, pltpu.SemaphoreType.DMA],
        ],
        compiler_params=pltpu.CompilerParams(use_tc_tiling_on_sc=False),
    )(words, table)


# speedup vs baseline: 1.1634x; 1.1634x over previous
import jax
import jax.numpy as jnp
from jax import lax
from jax.experimental import pallas as pl
from jax.experimental.pallas import tpu as pltpu
from jax.experimental.pallas import tpu_sc as plsc
from jax.experimental.layout import Format, Layout, with_layout_constraint

BATCH = 4096
HIST = 200
EMBED_DIM = 64

NC = 2
NS = 16
NW = NC * NS
ROWS_W = BATCH // NW


def _gather_body(words_hbm, table_hbm, out_hbm, idx_v, rows_v, gsems, osems):
    wid = lax.axis_index("s") * NC + lax.axis_index("c")
    base = wid * ROWS_W
    pltpu.sync_copy(words_hbm.at[pl.ds(base, ROWS_W)], idx_v)

    def fire_gather(i, h):
        pltpu.async_copy(
            table_hbm.at[idx_v.at[i]],
            rows_v.at[h],
            gsems[h],
        )

    def drain_gather(h):
        pltpu.make_async_copy(
            table_hbm.at[pl.ds(0, HIST)], rows_v.at[h], gsems[h]
        ).wait()

    def fire_write(i, h):
        pltpu.async_copy(rows_v.at[h], out_hbm.at[base + i], osems[h])

    def drain_write(h):
        pltpu.make_async_copy(rows_v.at[h], out_hbm.at[base], osems[h]).wait()

    fire_gather(0, 0)

    def body(t, _):
        for h in (0, 1):
            i = 2 * t + h
            @pl.when(jnp.logical_and(i >= 1, i + 1 < ROWS_W))
            def _():
                drain_write(1 - h)

            @pl.when(i + 1 < ROWS_W)
            def _():
                fire_gather(i + 1, 1 - h)

            drain_gather(h)
            fire_write(i, h)
        return ()

    lax.fori_loop(0, ROWS_W // 2, body, (), unroll=False)
    drain_write(0)
    drain_write(1)


def kernel(words, table):
    mesh = plsc.VectorSubcoreMesh(core_axis_name="c", subcore_axis_name="s")
    out = pl.kernel(
        _gather_body,
        out_type=jax.ShapeDtypeStruct((BATCH, HIST, EMBED_DIM), jnp.float32),
        mesh=mesh,
        scratch_types=[
            pltpu.VMEM((ROWS_W, HIST), jnp.int32),
            pltpu.VMEM((2, HIST, EMBED_DIM), jnp.float32),
            [pltpu.SemaphoreType.DMA, pltpu.SemaphoreType.DMA],
            [pltpu.SemaphoreType.DMA, pltpu.SemaphoreType.DMA],
        ],
        compiler_params=pltpu.CompilerParams(use_tc_tiling_on_sc=False),
    )(words, table)
    return with_layout_constraint(out, Layout(major_to_minor=(0, 1, 2)))
